# Initial kernel scaffold; baseline (speedup 1.0000x reference)
import jax
import jax.numpy as jnp
from jax.experimental import pallas as pl


def kernel(x, embed):
    raise NotImplementedError("write your pallas kernel here")


# speedup vs baseline: 1.1632x; 1.1632x over previous
import functools

import jax
import jax.numpy as jnp
from jax import lax
from jax.experimental import pallas as pl
from jax.experimental.pallas import tpu as pltpu
from jax.experimental.pallas import tpu_sc as plsc

K = 8192
D = 32


_ROWS = 256


def _first_argmax(d, base):
    m = jnp.max(d, axis=-1)
    iota = jax.lax.broadcasted_iota(jnp.int32, d.shape, 1)
    hit = d == m[:, None]
    i = jnp.min(jnp.where(hit, iota, K), axis=-1) + base
    return m, i


def _argmin_body(x_ref, et_ref, out_ref):
    x = x_ref[...]
    et = et_ref[...]
    mm = jnp.dot(x.astype(jnp.bfloat16), et.astype(jnp.bfloat16),
                 preferred_element_type=jnp.float32)
    xsq = jnp.sum(x * x, axis=1, keepdims=True)
    esq = jnp.sum(et * et, axis=0, keepdims=True)
    dist = -((xsq - 2.0 * mm) + esq)
    h = K // 2
    m1, i1 = _first_argmax(dist[:, :h], 0)
    m2, i2 = _first_argmax(dist[:, h:], h)
    s1 = m1.astype(jnp.bfloat16).astype(jnp.float32)
    idx = jnp.where(m2 > s1, i2, i1)
    out_ref[0, 0, :] = idx.astype(jnp.int32)


def _argmin_tc(flat, et):
    n = flat.shape[0]
    nb = n // _ROWS
    out = pl.pallas_call(
        _argmin_body,
        grid=(nb,),
        in_specs=[
            pl.BlockSpec((_ROWS, D), lambda i: (i, 0)),
            pl.BlockSpec((D, K), lambda i: (0, 0)),
        ],
        out_specs=pl.BlockSpec((1, 1, _ROWS), lambda i: (i, 0, 0)),
        out_shape=jax.ShapeDtypeStruct((nb, 1, _ROWS), jnp.int32),
    )(flat, et)
    return out.reshape(-1)


_NC = 2
_NS = 16
_NW = _NC * _NS
_CHUNK = 128


def _make_gather(B):
    b_per_w = B // _NW
    n_chunks = b_per_w // _CHUNK
    mesh = plsc.VectorSubcoreMesh(core_axis_name="c", subcore_axis_name="s")

    @functools.partial(
        pl.kernel, mesh=mesh,
        compiler_params=pltpu.CompilerParams(use_tc_tiling_on_sc=False),
        out_type=jax.ShapeDtypeStruct((B, D), jnp.float32),
        scratch_types=[
            pltpu.VMEM((n_chunks, _CHUNK), jnp.int32),
            pltpu.VMEM((b_per_w, D), jnp.float32),
            pltpu.SemaphoreType.DMA,
        ],
    )
    def gather(table_hbm, idx_hbm, out_hbm, idx_v, rows_v, sem):
        wid = lax.axis_index("s") * _NC + lax.axis_index("c")
        base = wid * b_per_w
        pltpu.sync_copy(idx_hbm.at[pl.ds(wid * n_chunks, n_chunks)], idx_v)
        handles = []
        for c in range(n_chunks):
            handles.append(pltpu.async_copy(
                table_hbm.at[idx_v.at[c]],
                rows_v.at[pl.ds(c * _CHUNK, _CHUNK)],
                sem))
        for h in handles:
            h.wait()
        pltpu.sync_copy(rows_v, out_hbm.at[pl.ds(base, b_per_w)])

    return gather


def kernel(x, embed):
    shape = x.shape
    flat = x.reshape(-1, shape[-1])
    et = embed.T
    idx_flat = _argmin_tc(flat, et)
    n = flat.shape[0]
    idx2 = idx_flat.reshape(n // _CHUNK, _CHUNK)
    quant = _make_gather(n)(embed, idx2)
    return quant.reshape(shape), idx_flat.reshape(shape[:-1])
